# speedup vs baseline: 2.5774x; 2.5774x over previous
import jax
import jax.numpy as jnp
from jax import lax
from jax.experimental import pallas as pl
from jax.experimental.pallas import tpu as pltpu
from jax.experimental.pallas import tpu_sc as plsc

_P = 128
_C = 384
_B = 8
_NW = 32
_CPW = _C * _B // _NW
_LANES = 16


def _body(prefix_hbm, table_hbm, out_hbm, pref_v, idx_v, tile_v,
          gsem0, gsem1, wsem0, wsem1):
    wid = lax.axis_index("s") * 2 + lax.axis_index("c")
    b = wid // 4
    c0 = (wid % 4) * _CPW
    base = wid * _CPW

    gsem = (gsem0, gsem1)
    wsem = (wsem0, wsem1)

    pltpu.sync_copy(prefix_hbm.at[b], pref_v)

    @pl.loop(0, _CPW)
    def _compute_idx(j):
        for k in range(_P // _LANES):
            sl = pl.ds(k * _LANES, _LANES)
            idx_v[j, sl] = pref_v[sl] * _C + (c0 + j)

    pltpu.async_copy(table_hbm.at[idx_v.at[0]], tile_v.at[0], gsem[0])

    @pl.loop(0, _CPW, step=2)
    def _main(g):
        for t in range(2):
            j = g + t
            tn = 1 - t
            jn = j + 1

            @pl.when(jnp.logical_and(jn < _CPW, j >= 1))
            def _wait_prev_write():
                pltpu.make_async_copy(tile_v.at[tn], out_hbm.at[base],
                                      wsem[tn]).wait()

            @pl.when(jn < _CPW)
            def _start_next_gather():
                pltpu.async_copy(table_hbm.at[idx_v.at[jn]], tile_v.at[tn],
                                 gsem[tn])

            pltpu.make_async_copy(table_hbm.at[idx_v.at[j]], tile_v.at[t],
                                  gsem[t]).wait()
            pltpu.async_copy(tile_v.at[t], out_hbm.at[base + j], wsem[t])

    pltpu.make_async_copy(tile_v.at[0], out_hbm.at[base], wsem[0]).wait()
    pltpu.make_async_copy(tile_v.at[1], out_hbm.at[base], wsem[1]).wait()


def kernel(prefix, table):
    prefix = prefix.astype(jnp.int32)
    table2 = table.reshape(_P * _C, _P)

    mesh = plsc.VectorSubcoreMesh(core_axis_name="c", subcore_axis_name="s")
    fn = pl.kernel(
        _body,
        out_type=jax.ShapeDtypeStruct((_B * _C, _P, _P), jnp.float32),
        mesh=mesh,
        scratch_types=[
            pltpu.VMEM((_P,), jnp.int32),
            pltpu.VMEM((_CPW, _P), jnp.int32),
            pltpu.VMEM((2, _P, _P), jnp.float32),
            pltpu.SemaphoreType.DMA,
            pltpu.SemaphoreType.DMA,
            pltpu.SemaphoreType.DMA,
            pltpu.SemaphoreType.DMA,
        ],
    )
    out = fn(prefix, table2)
    return out.reshape(_B, 48, 8, _P, _P)


# speedup vs baseline: 2.7499x; 1.0669x over previous
import jax
import jax.numpy as jnp
from jax import lax
from jax.experimental import pallas as pl
from jax.experimental.pallas import tpu as pltpu
from jax.experimental.pallas import tpu_sc as plsc

_P = 128
_C = 384
_B = 8
_NW = 32
_CPW = _C * _B // _NW
_LANES = 16
_NBUF = 4


def _body(prefix_hbm, table_hbm, out_hbm, pref_v, idx_v, tile_v,
          gsem0, gsem1, gsem2, gsem3, wsem0, wsem1, wsem2, wsem3):
    wid = lax.axis_index("s") * 2 + lax.axis_index("c")
    b = wid // 4
    c0 = (wid % 4) * _CPW
    base = wid * _CPW

    gsem = (gsem0, gsem1, gsem2, gsem3)
    wsem = (wsem0, wsem1, wsem2, wsem3)

    pltpu.sync_copy(prefix_hbm.at[b], pref_v)

    @pl.loop(0, _CPW)
    def _compute_idx(j):
        for k in range(_P // _LANES):
            sl = pl.ds(k * _LANES, _LANES)
            idx_v[j, sl] = pref_v[sl] * _C + (c0 + j)

    for t in range(3):
        pltpu.async_copy(table_hbm.at[idx_v.at[t]], tile_v.at[t], gsem[t])

    @pl.loop(0, _CPW, step=_NBUF)
    def _main(g):
        for t in range(_NBUF):
            j = g + t
            tg = (t + 3) % _NBUF
            jn = j + 3

            pltpu.make_async_copy(table_hbm.at[idx_v.at[0]], tile_v.at[t],
                                  gsem[t]).wait()
            pltpu.async_copy(tile_v.at[t], out_hbm.at[base + j], wsem[t])

            @pl.when(jnp.logical_and(jn < _CPW, j >= 1))
            def _wait_prev_write():
                pltpu.make_async_copy(tile_v.at[tg], out_hbm.at[base],
                                      wsem[tg]).wait()

            @pl.when(jn < _CPW)
            def _start_next_gather():
                pltpu.async_copy(table_hbm.at[idx_v.at[jn]], tile_v.at[tg],
                                 gsem[tg])

    for t in range(_NBUF):
        pltpu.make_async_copy(tile_v.at[t], out_hbm.at[base], wsem[t]).wait()


def kernel(prefix, table):
    prefix = prefix.astype(jnp.int32)
    table2 = table.reshape(_P * _C, _P)

    mesh = plsc.VectorSubcoreMesh(core_axis_name="c", subcore_axis_name="s")
    fn = pl.kernel(
        _body,
        out_type=jax.ShapeDtypeStruct((_B * _C, _P, _P), jnp.float32),
        mesh=mesh,
        scratch_types=[
            pltpu.VMEM((_P,), jnp.int32),
            pltpu.VMEM((_CPW, _P), jnp.int32),
            pltpu.VMEM((_NBUF, _P, _P), jnp.float32),
        ] + [pltpu.SemaphoreType.DMA] * (2 * _NBUF),
    )
    out = fn(prefix, table2)
    return out.reshape(_B, 48, 8, _P, _P)


# speedup vs baseline: 4.0260x; 1.4641x over previous
import jax
import jax.numpy as jnp
from jax import lax
from jax.experimental import pallas as pl
from jax.experimental.pallas import tpu as pltpu
from jax.experimental.pallas import tpu_sc as plsc

_P = 128
_C = 384
_B = 8
_LANES = 16
_NBUF = 4
_W = 48
_TPP = 24


def _body(prefix_hbm, table3_hbm, out_hbm, pref_v, idx_v, tile_v, slab_sh,
          ssem, gsem0, gsem1, gsem2, gsem3, wsem0, wsem1, wsem2, wsem3):
    core = lax.axis_index("c")
    sub = lax.axis_index("s")
    b = sub // 2
    half = sub % 2

    gsem = (gsem0, gsem1, gsem2, gsem3)
    wsem = (wsem0, wsem1, wsem2, wsem3)

    pltpu.sync_copy(prefix_hbm.at[b], pref_v)

    @pl.loop(0, _TPP)
    def _compute_idx(j):
        for k in range(_P // _LANES):
            sl = pl.ds(k * _LANES, _LANES)
            idx_v[j, sl] = pref_v[sl] + (half * _TPP + j) * _P

    @pl.loop(0, _C // _W // 2)
    def _phase(ph):
        c0 = core * (_C // 2) + ph * _W

        for m in range(_W // 16):
            cc = sub + 16 * m
            pltpu.async_copy(table3_hbm.at[:, c0 + cc, :],
                             slab_sh.at[pl.ds(cc * _P, _P)], ssem)
        for m in range(_W // 16):
            pltpu.make_async_copy(table3_hbm.at[:, 0, :],
                                  slab_sh.at[pl.ds(0, _P)], ssem).wait()
        plsc.subcore_barrier()

        base = b * _C + c0 + half * _TPP

        for t in range(3):
            pltpu.async_copy(slab_sh.at[idx_v.at[t]], tile_v.at[t], gsem[t])

        @pl.loop(0, _TPP, step=_NBUF)
        def _main(g):
            for t in range(_NBUF):
                j = g + t
                tg = (t + 3) % _NBUF
                jn = j + 3

                pltpu.make_async_copy(slab_sh.at[idx_v.at[0]], tile_v.at[t],
                                      gsem[t]).wait()
                pltpu.async_copy(tile_v.at[t], out_hbm.at[base + j], wsem[t])

                @pl.when(jnp.logical_and(jn < _TPP, j >= 1))
                def _wait_prev_write():
                    pltpu.make_async_copy(tile_v.at[tg], out_hbm.at[base],
                                          wsem[tg]).wait()

                @pl.when(jn < _TPP)
                def _start_next_gather():
                    pltpu.async_copy(slab_sh.at[idx_v.at[jn]], tile_v.at[tg],
                                     gsem[tg])

        for t in range(_NBUF):
            pltpu.make_async_copy(tile_v.at[t], out_hbm.at[base],
                                  wsem[t]).wait()
        plsc.subcore_barrier()


def kernel(prefix, table):
    prefix = prefix.astype(jnp.int32)
    table3 = table.reshape(_P, _C, _P)

    mesh = plsc.VectorSubcoreMesh(core_axis_name="c", subcore_axis_name="s")
    fn = pl.kernel(
        _body,
        out_type=jax.ShapeDtypeStruct((_B * _C, _P, _P), jnp.float32),
        mesh=mesh,
        scratch_types=[
            pltpu.VMEM((_P,), jnp.int32),
            pltpu.VMEM((_TPP, _P), jnp.int32),
            pltpu.VMEM((_NBUF, _P, _P), jnp.float32),
            pltpu.VMEM_SHARED((_W * _P, _P), jnp.float32),
        ] + [pltpu.SemaphoreType.DMA] * (1 + 2 * _NBUF),
    )
    out = fn(prefix, table3)
    return out.reshape(_B, 48, 8, _P, _P)


# speedup vs baseline: 4.7547x; 1.1810x over previous
import jax
import jax.numpy as jnp
from jax import lax
from jax.experimental import pallas as pl
from jax.experimental.pallas import tpu as pltpu
from jax.experimental.pallas import tpu_sc as plsc

_P = 128
_C = 384
_B = 8
_LANES = 16
_NBUF = 4
_W = 48
_TPP = 24


def _body(prefix_hbm, table_hbm, out_hbm, pref_v, idx_v, tile_v, slab_sh,
          ssem, gsem0, gsem1, gsem2, gsem3, wsem0, wsem1, wsem2, wsem3):
    core = lax.axis_index("c")
    sub = lax.axis_index("s")
    b = sub // 2
    half = sub % 2

    gsem = (gsem0, gsem1, gsem2, gsem3)
    wsem = (wsem0, wsem1, wsem2, wsem3)

    pltpu.sync_copy(prefix_hbm.at[b], pref_v)

    @pl.loop(0, _TPP)
    def _compute_idx(j):
        for k in range(_P // _LANES):
            sl = pl.ds(k * _LANES, _LANES)
            idx_v[j, sl] = pref_v[sl] + (half * _TPP + j) * _P

    @pl.loop(0, _C // _W // 2)
    def _phase(ph):
        c0 = core * (_C // 2) + ph * _W

        for m in range(_W // 16):
            cc = sub + 16 * m
            pltpu.async_copy(table_hbm.at[:, pl.ds((c0 + cc) * _P, _P)],
                             slab_sh.at[pl.ds(cc * _P, _P)], ssem)
        for m in range(_W // 16):
            pltpu.make_async_copy(table_hbm.at[:, pl.ds(0, _P)],
                                  slab_sh.at[pl.ds(0, _P)], ssem).wait()
        plsc.subcore_barrier()

        base = b * _C + c0 + half * _TPP

        for t in range(3):
            pltpu.async_copy(slab_sh.at[idx_v.at[t]], tile_v.at[t], gsem[t])

        @pl.loop(0, _TPP, step=_NBUF)
        def _main(g):
            for t in range(_NBUF):
                j = g + t
                tg = (t + 3) % _NBUF
                jn = j + 3

                pltpu.make_async_copy(slab_sh.at[idx_v.at[0]], tile_v.at[t],
                                      gsem[t]).wait()
                pltpu.async_copy(tile_v.at[t], out_hbm.at[base + j], wsem[t])

                @pl.when(jnp.logical_and(jn < _TPP, j >= 1))
                def _wait_prev_write():
                    pltpu.make_async_copy(tile_v.at[tg], out_hbm.at[base],
                                          wsem[tg]).wait()

                @pl.when(jn < _TPP)
                def _start_next_gather():
                    pltpu.async_copy(slab_sh.at[idx_v.at[jn]], tile_v.at[tg],
                                     gsem[tg])

        for t in range(_NBUF):
            pltpu.make_async_copy(tile_v.at[t], out_hbm.at[base],
                                  wsem[t]).wait()
        plsc.subcore_barrier()


def kernel(prefix, table):
    prefix = prefix.astype(jnp.int32)

    mesh = plsc.VectorSubcoreMesh(core_axis_name="c", subcore_axis_name="s")
    fn = pl.kernel(
        _body,
        out_type=jax.ShapeDtypeStruct((_B * _C, _P, _P), jnp.float32),
        mesh=mesh,
        scratch_types=[
            pltpu.VMEM((_P,), jnp.int32),
            pltpu.VMEM((_TPP, _P), jnp.int32),
            pltpu.VMEM((_NBUF, _P, _P), jnp.float32),
            pltpu.VMEM_SHARED((_W * _P, _P), jnp.float32),
        ] + [pltpu.SemaphoreType.DMA] * (1 + 2 * _NBUF),
    )
    out = fn(prefix, table)
    return out.reshape(_B, 48, 8, _P, _P)


# speedup vs baseline: 5.2331x; 1.1006x over previous
import jax
import jax.numpy as jnp
from jax import lax
from jax.experimental import pallas as pl
from jax.experimental.pallas import tpu as pltpu
from jax.experimental.pallas import tpu_sc as plsc

_P = 128
_C = 384
_B = 8
_LANES = 16
_NBUF = 4
_W = 24
_NPH = _C // 2 // _W
_TPP = 12


def _stage_slab(table_hbm, slab, c0, sub, ssem):
    pltpu.async_copy(table_hbm.at[:, pl.ds((c0 + sub) * _P, _P)],
                     slab.at[pl.ds(sub * _P, _P)], ssem)

    @pl.when(sub < _W - 16)
    def _second():
        pltpu.async_copy(table_hbm.at[:, pl.ds((c0 + 16 + sub) * _P, _P)],
                         slab.at[pl.ds((16 + sub) * _P, _P)], ssem)


def _drain_slab(table_hbm, slab, sub, ssem):
    pltpu.make_async_copy(table_hbm.at[:, pl.ds(0, _P)],
                          slab.at[pl.ds(0, _P)], ssem).wait()

    @pl.when(sub < _W - 16)
    def _second():
        pltpu.make_async_copy(table_hbm.at[:, pl.ds(0, _P)],
                              slab.at[pl.ds(0, _P)], ssem).wait()


def _body(prefix_hbm, table_hbm, out_hbm, pref_v, idx_v, tile_v,
          slab_a, slab_b, ssem,
          gsem0, gsem1, gsem2, gsem3, wsem0, wsem1, wsem2, wsem3):
    core = lax.axis_index("c")
    sub = lax.axis_index("s")
    b = sub // 2
    half = sub % 2

    gsem = (gsem0, gsem1, gsem2, gsem3)
    wsem = (wsem0, wsem1, wsem2, wsem3)
    slabs = (slab_a, slab_b)
    cbase = core * (_C // 2)

    _stage_slab(table_hbm, slab_a, cbase, sub, ssem)

    pltpu.sync_copy(prefix_hbm.at[b], pref_v)

    @pl.loop(0, _TPP)
    def _compute_idx(j):
        for k in range(_P // _LANES):
            sl = pl.ds(k * _LANES, _LANES)
            idx_v[j, sl] = pref_v[sl] + (half * _TPP + j) * _P

    for ph in range(_NPH):
        slab = slabs[ph % 2]
        c0 = cbase + ph * _W

        _drain_slab(table_hbm, slab, sub, ssem)
        plsc.subcore_barrier()

        if ph + 1 < _NPH:
            _stage_slab(table_hbm, slabs[(ph + 1) % 2], c0 + _W, sub, ssem)

        base = b * _C + c0 + half * _TPP

        for t in range(3):
            pltpu.async_copy(slab.at[idx_v.at[t]], tile_v.at[t], gsem[t])

        @pl.loop(0, _TPP, step=_NBUF)
        def _main(g):
            for t in range(_NBUF):
                j = g + t
                tg = (t + 3) % _NBUF
                jn = j + 3

                pltpu.make_async_copy(slab.at[idx_v.at[0]], tile_v.at[t],
                                      gsem[t]).wait()
                pltpu.async_copy(tile_v.at[t], out_hbm.at[base + j], wsem[t])

                @pl.when(jnp.logical_and(jn < _TPP, j >= 1))
                def _wait_prev_write():
                    pltpu.make_async_copy(tile_v.at[tg], out_hbm.at[base],
                                          wsem[tg]).wait()

                @pl.when(jn < _TPP)
                def _start_next_gather():
                    pltpu.async_copy(slab.at[idx_v.at[jn]], tile_v.at[tg],
                                     gsem[tg])

        for t in range(_NBUF):
            pltpu.make_async_copy(tile_v.at[t], out_hbm.at[base],
                                  wsem[t]).wait()


def kernel(prefix, table):
    prefix = prefix.astype(jnp.int32)

    mesh = plsc.VectorSubcoreMesh(core_axis_name="c", subcore_axis_name="s")
    fn = pl.kernel(
        _body,
        out_type=jax.ShapeDtypeStruct((_B * _C, _P, _P), jnp.float32),
        mesh=mesh,
        scratch_types=[
            pltpu.VMEM((_P,), jnp.int32),
            pltpu.VMEM((_TPP, _P), jnp.int32),
            pltpu.VMEM((_NBUF, _P, _P), jnp.float32),
            pltpu.VMEM_SHARED((_W * _P, _P), jnp.float32),
            pltpu.VMEM_SHARED((_W * _P, _P), jnp.float32),
        ] + [pltpu.SemaphoreType.DMA] * (1 + 2 * _NBUF),
    )
    out = fn(prefix, table)
    return out.reshape(_B, 48, 8, _P, _P)


# speedup vs baseline: 5.5384x; 1.0583x over previous
import jax
import jax.numpy as jnp
from jax import lax
from jax.experimental import pallas as pl
from jax.experimental.pallas import tpu as pltpu
from jax.experimental.pallas import tpu_sc as plsc

_P = 128
_C = 384
_B = 8
_LANES = 16
_NBUF = 4
_W = 24
_NPH = _C // 2 // _W
_TPP = 12


def _stage_slab(table_hbm, slab, c0, sub, ssem):
    pltpu.async_copy(table_hbm.at[:, pl.ds((c0 + sub) * _P, _P)],
                     slab.at[pl.ds(sub * _P, _P)], ssem)

    @pl.when(sub < _W - 16)
    def _second():
        pltpu.async_copy(table_hbm.at[:, pl.ds((c0 + 16 + sub) * _P, _P)],
                         slab.at[pl.ds((16 + sub) * _P, _P)], ssem)


def _drain_slab(table_hbm, slab, sub, ssem):
    pltpu.make_async_copy(table_hbm.at[:, pl.ds(0, _P)],
                          slab.at[pl.ds(0, _P)], ssem).wait()

    @pl.when(sub < _W - 16)
    def _second():
        pltpu.make_async_copy(table_hbm.at[:, pl.ds(0, _P)],
                              slab.at[pl.ds(0, _P)], ssem).wait()


def _body(prefix_hbm, table_hbm, out_hbm, pref_v, idx_v, tile_v,
          slab_a, slab_b, ssem,
          gsem0, gsem1, gsem2, gsem3, wsem0, wsem1, wsem2, wsem3):
    core = lax.axis_index("c")
    sub = lax.axis_index("s")
    b = sub // 2
    half = sub % 2

    gsem = (gsem0, gsem1, gsem2, gsem3)
    wsem = (wsem0, wsem1, wsem2, wsem3)
    slabs = (slab_a, slab_b)
    cbase = core * (_C // 2)

    _stage_slab(table_hbm, slab_a, cbase, sub, ssem)

    pltpu.sync_copy(prefix_hbm.at[b], pref_v)

    @pl.loop(0, _TPP)
    def _compute_idx(j):
        for k in range(_P // _LANES):
            sl = pl.ds(k * _LANES, _LANES)
            idx_v[j, sl] = pref_v[sl] + (half * _TPP + j) * _P

    for ph in range(_NPH):
        slab = slabs[ph % 2]
        c0 = cbase + ph * _W

        _drain_slab(table_hbm, slab, sub, ssem)
        plsc.subcore_barrier()

        if ph + 1 < _NPH:
            _stage_slab(table_hbm, slabs[(ph + 1) % 2], c0 + _W, sub, ssem)

        base = b * _C + c0 + half * _TPP

        for t in range(3):
            if ph > 0:
                pltpu.make_async_copy(tile_v.at[t], out_hbm.at[base],
                                      wsem[t]).wait()
            pltpu.async_copy(slab.at[idx_v.at[t]], tile_v.at[t], gsem[t])

        @pl.loop(0, _TPP, step=_NBUF)
        def _main(g):
            for t in range(_NBUF):
                j = g + t
                tg = (t + 3) % _NBUF
                jn = j + 3

                pltpu.make_async_copy(slab.at[idx_v.at[0]], tile_v.at[t],
                                      gsem[t]).wait()
                pltpu.async_copy(tile_v.at[t], out_hbm.at[base + j], wsem[t])

                if ph > 0:
                    wait_cond = jn < _TPP
                else:
                    wait_cond = jnp.logical_and(jn < _TPP, j >= 1)

                @pl.when(wait_cond)
                def _wait_prev_write():
                    pltpu.make_async_copy(tile_v.at[tg], out_hbm.at[base],
                                          wsem[tg]).wait()

                @pl.when(jn < _TPP)
                def _start_next_gather():
                    pltpu.async_copy(slab.at[idx_v.at[jn]], tile_v.at[tg],
                                     gsem[tg])

    for t in range(_NBUF):
        pltpu.make_async_copy(tile_v.at[t], out_hbm.at[0], wsem[t]).wait()


def kernel(prefix, table):
    prefix = prefix.astype(jnp.int32)

    mesh = plsc.VectorSubcoreMesh(core_axis_name="c", subcore_axis_name="s")
    fn = pl.kernel(
        _body,
        out_type=jax.ShapeDtypeStruct((_B * _C, _P, _P), jnp.float32),
        mesh=mesh,
        scratch_types=[
            pltpu.VMEM((_P,), jnp.int32),
            pltpu.VMEM((_TPP, _P), jnp.int32),
            pltpu.VMEM((_NBUF, _P, _P), jnp.float32),
            pltpu.VMEM_SHARED((_W * _P, _P), jnp.float32),
            pltpu.VMEM_SHARED((_W * _P, _P), jnp.float32),
        ] + [pltpu.SemaphoreType.DMA] * (1 + 2 * _NBUF),
    )
    out = fn(prefix, table)
    return out.reshape(_B, 48, 8, _P, _P)
